# baseline (speedup 1.0000x reference)
import jax
import jax.numpy as jnp
from jax import lax
from jax.experimental import pallas as pl
from jax.experimental.pallas import tpu as pltpu
from jax.experimental.pallas import tpu_sc as plsc

_B, _A, _H, _W = 32, 5, 26, 26
_NT = 50
_NTP = 64
_TPAD = 256
_NPAIR = _A * _H
_PER = 9
_THRESH = 0.5


def _rsqrt(v):
    i = plsc.bitcast(v, jnp.int32)
    y = plsc.bitcast(jnp.int32(0x5F3759DF) - (i >> 1), jnp.float32)
    for _ in range(3):
        y = y * (1.5 - 0.5 * v * y * y)
    return jnp.where(v == 0.0, jnp.float32(jnp.inf), y)


def _body(xt_hbm, anc_hbm, tg_hbm, res_hbm,
          tbuf, anc_v, cells, keptv, flagv, wmr, pcr,
          g0r, g1r, g2r, g3r, table, accv, sumbuf, blk,
          lcode, lwgt, lg0, lg1, lg2, lg3,
          s_code, s_wgt, s_g0, s_g1, s_g2, s_g3, s_part, sem):
    cid = lax.axis_index("c")
    sid = lax.axis_index("s")
    b = cid * 16 + sid

    lo = sid * _PER
    boff = pl.multiple_of(cid * 16, 8)
    blk_cps = []
    for j in range(_PER):
        p = jnp.minimum(lo + j, jnp.int32(_NPAIR - 1))
        a_s = ((p >= _H).astype(jnp.int32) + (p >= 2 * _H) + (p >= 3 * _H)
               + (p >= 4 * _H))
        h_s = p - a_s * _H
        blk_cps.append(
            pltpu.async_copy(
                xt_hbm.at[a_s, h_s, pl.ds(0, 4), pl.ds(boff, 16)],
                blk.at[j], sem))

    tstart = b * (_NT * 5)
    tshift = tstart & 7
    tastart = pl.multiple_of(tstart - tshift, 8)
    pltpu.sync_copy(tg_hbm.at[pl.ds(tastart, _TPAD)], tbuf)
    pltpu.sync_copy(anc_hbm, anc_v.at[pl.ds(0, 2 * _A)])

    lane = lax.iota(jnp.int32, 16)
    av = anc_v[...]

    for ci in range(4):
        tvec = lane + 16 * ci
        valid = tvec < _NT
        trow = jnp.minimum(tvec, _NT - 1) * 5 + tshift
        x = plsc.load_gather(tbuf, [trow + 1])
        y = plsc.load_gather(tbuf, [trow + 2])
        w = plsc.load_gather(tbuf, [trow + 3])
        h = plsc.load_gather(tbuf, [trow + 4])
        x = jnp.where(valid, x, 0.0)
        y = jnp.where(valid, y, 0.0)
        w = jnp.where(valid, w, 0.0)
        h = jnp.where(valid, h, 0.0)
        kept = valid & ~((x == 0.0) & (y == 0.0) & (w == 0.0) & (h == 0.0))

        cxf = x * float(_W)
        cyf = y * float(_H)
        cx = cxf.astype(jnp.int32)
        cy = cyf.astype(jnp.int32)
        ctx = cxf - cx.astype(jnp.float32) - 0.5
        cty = cyf - cy.astype(jnp.float32) - 0.5
        tw = w * float(_W)
        th = h * float(_H)
        t_area = tw * th

        best = jnp.full((16,), -1.0, jnp.float32)
        bidx = jnp.zeros((16,), jnp.int32)
        tx0 = ctx - tw * 0.5
        tx1 = ctx + tw * 0.5
        ty0 = cty - th * 0.5
        ty1 = cty + th * 0.5
        for a in range(_A):
            aw = av[2 * a]
            ah = av[2 * a + 1]
            aw2 = aw * 0.5
            ah2 = ah * 0.5
            x0 = jnp.maximum(tx0, -aw2)
            x1 = jnp.minimum(tx1, aw2)
            y0 = jnp.maximum(ty0, -ah2)
            y1 = jnp.minimum(ty1, ah2)
            ivl = (x0 < x1) & (y0 < y1)
            inter = jnp.where(ivl, (x1 - x0) * (y1 - y0), 0.0)
            iou = inter / (t_area + aw * ah - inter)
            upd = iou > best
            best = jnp.where(upd, iou, best)
            bidx = jnp.where(upd, jnp.int32(a), bidx)

        flagged = kept & (best > _THRESH)
        cell = cy * _W + cx
        sl = pl.ds(16 * ci, 16)
        cells[sl] = cell
        keptv[sl] = kept.astype(jnp.int32)
        flagv[sl] = flagged.astype(jnp.int32)
        i_row = cell * _A + bidx
        a2 = ((i_row.astype(jnp.float32) + 0.5)
              * (1.0 / float(_H * _W))).astype(jnp.int32)
        rem = i_row - a2 * (_H * _W)
        h2 = ((rem.astype(jnp.float32) + 0.5)
              * (1.0 / float(_W))).astype(jnp.int32)
        w2 = rem - h2 * _W
        pcr[sl] = (a2 * _H + h2) * 32 + w2
        g0r[sl] = cxf
        g1r[sl] = cyf
        g2r[sl] = tw
        g3r[sl] = th

    for ci in range(4):
        sl = pl.ds(16 * ci, 16)
        cvec = cells[sl]
        kvec = keptv[sl]
        addrs = jnp.where(kvec > 0, cvec, jnp.int32(_H * _W))
        tvec = lane + 16 * ci
        nt_here = min(16, _NT - 16 * ci)
        for j in range(nt_here):
            plsc.store_scatter(table, [addrs], tvec, mask=lane == j)

    cnt = jnp.zeros((16,), jnp.float32)
    for ci in range(4):
        sl = pl.ds(16 * ci, 16)
        tvec = lane + 16 * ci
        winner = plsc.load_gather(table, [cells[sl]])
        wm = (flagv[sl] > 0) & (winner == tvec)
        wmr[sl] = wm.astype(jnp.int32)
        cnt = cnt + jnp.where(wm, 1.0, 0.0)
    n2 = jnp.sum(cnt)
    recip = 1.0 / (2.0 * (jnp.zeros((16,), jnp.float32) + n2))
    for ci in range(4):
        sl = pl.ds(16 * ci, 16)
        wm = wmr[sl] > 0
        pcr[sl] = jnp.where(wm, pcr[sl], 0)
        flagv[sl] = plsc.bitcast(jnp.where(wm, recip, 0.0), jnp.int32)

    psl = pl.ds(sid * _NTP, _NTP)
    pltpu.sync_copy(pcr, s_code.at[psl])
    pltpu.sync_copy(flagv, s_wgt.at[psl])
    pltpu.sync_copy(g0r, s_g0.at[psl])
    pltpu.sync_copy(g1r, s_g1.at[psl])
    pltpu.sync_copy(g2r, s_g2.at[psl])
    pltpu.sync_copy(g3r, s_g3.at[psl])
    plsc.subcore_barrier()

    for cp in blk_cps:
        cp.wait()

    pltpu.sync_copy(s_code, lcode)
    pltpu.sync_copy(s_wgt, lwgt)
    pltpu.sync_copy(s_g0, lg0)
    pltpu.sync_copy(s_g1, lg1)
    pltpu.sync_copy(s_g2, lg2)
    pltpu.sync_copy(s_g3, lg3)

    contrib = jnp.zeros((16,), jnp.float32)
    for r in range(64):
        sl = pl.ds(16 * r, 16)
        code = lcode[sl]
        wgt = plsc.bitcast(lwgt[sl], jnp.float32)
        pairv = code >> 5
        cxv = code & 31
        rel = pairv - lo
        mine = (wgt > 0.0) & (rel >= 0) & (rel < _PER)
        relc = jnp.clip(rel, 0, _PER - 1)
        brel = (lane + 16 * r) >> 6
        p0 = plsc.load_gather(blk, [relc, jnp.full((16,), 0, jnp.int32),
                                    brel, cxv])
        p1 = plsc.load_gather(blk, [relc, jnp.full((16,), 1, jnp.int32),
                                    brel, cxv])
        p2 = plsc.load_gather(blk, [relc, jnp.full((16,), 2, jnp.int32),
                                    brel, cxv])
        p3 = plsc.load_gather(blk, [relc, jnp.full((16,), 3, jnp.int32),
                                    brel, cxv])
        d0 = p0 - lg0[sl]
        d1 = p1 - lg1[sl]
        d2 = _rsqrt(p2) - _rsqrt(lg2[sl])
        d3 = _rsqrt(p3) - _rsqrt(lg3[sl])
        ssq = d0 * d0 + d1 * d1 + d2 * d2 + d3 * d3
        contrib = contrib + jnp.where(mine, wgt * ssq, 0.0)

    part = jnp.sum(contrib)
    accv[...] = jnp.where(lane == 0, part, 0.0)
    pltpu.sync_copy(accv, s_part.at[pl.ds(sid * 16, 16)])
    plsc.subcore_barrier()

    @pl.when(sid == 0)
    def _():
        pltpu.sync_copy(s_part, sumbuf)
        acc = sumbuf[pl.ds(0, 16)]
        for i in range(1, 16):
            acc = acc + sumbuf[pl.ds(16 * i, 16)]
        accv[...] = acc
        pltpu.sync_copy(accv, res_hbm.at[cid])


def kernel(output, anchors, targets):
    xt = output.transpose(1, 2, 4, 0, 3)
    tg1d = targets.reshape(-1)
    anc1d = anchors.reshape(-1)
    mesh = plsc.VectorSubcoreMesh(core_axis_name="c", subcore_axis_name="s")
    k = pl.kernel(
        _body,
        mesh=mesh,
        compiler_params=pltpu.CompilerParams(needs_layout_passes=False),
        out_type=jax.ShapeDtypeStruct((2, 16), jnp.float32),
        scratch_types=[
            pltpu.VMEM((_TPAD,), jnp.float32),
            pltpu.VMEM((16,), jnp.float32),
            pltpu.VMEM((_NTP,), jnp.int32),
            pltpu.VMEM((_NTP,), jnp.int32),
            pltpu.VMEM((_NTP,), jnp.int32),
            pltpu.VMEM((_NTP,), jnp.int32),
            pltpu.VMEM((_NTP,), jnp.int32),
            pltpu.VMEM((_NTP,), jnp.float32),
            pltpu.VMEM((_NTP,), jnp.float32),
            pltpu.VMEM((_NTP,), jnp.float32),
            pltpu.VMEM((_NTP,), jnp.float32),
            pltpu.VMEM((_H * _W + 8,), jnp.int32),
            pltpu.VMEM((16,), jnp.float32),
            pltpu.VMEM((256,), jnp.float32),
            pltpu.VMEM((_PER, 4, 16, _W), jnp.float32),
            pltpu.VMEM((16 * _NTP,), jnp.int32),
            pltpu.VMEM((16 * _NTP,), jnp.int32),
            pltpu.VMEM((16 * _NTP,), jnp.float32),
            pltpu.VMEM((16 * _NTP,), jnp.float32),
            pltpu.VMEM((16 * _NTP,), jnp.float32),
            pltpu.VMEM((16 * _NTP,), jnp.float32),
            pltpu.VMEM_SHARED((16 * _NTP,), jnp.int32),
            pltpu.VMEM_SHARED((16 * _NTP,), jnp.int32),
            pltpu.VMEM_SHARED((16 * _NTP,), jnp.float32),
            pltpu.VMEM_SHARED((16 * _NTP,), jnp.float32),
            pltpu.VMEM_SHARED((16 * _NTP,), jnp.float32),
            pltpu.VMEM_SHARED((16 * _NTP,), jnp.float32),
            pltpu.VMEM_SHARED((256,), jnp.float32),
            pltpu.SemaphoreType.DMA,
        ],
    )
    res = k(xt, anc1d, tg1d)
    return (res[0, 0] + res[1, 0]) / jnp.float32(_B)


# speedup vs baseline: 1.1126x; 1.1126x over previous
import jax
import jax.numpy as jnp
from jax import lax
from jax.experimental import pallas as pl
from jax.experimental.pallas import tpu as pltpu
from jax.experimental.pallas import tpu_sc as plsc

_B, _A, _H, _W = 32, 5, 26, 26
_NT = 50
_NTP = 64
_TPAD = 256
_NPAIR = _A * _H
_PER = 17
_THRESH = 0.5


def _rsqrt(v):
    i = plsc.bitcast(v, jnp.int32)
    y = plsc.bitcast(jnp.int32(0x5F3759DF) - (i >> 1), jnp.float32)
    for _ in range(3):
        y = y * (1.5 - 0.5 * v * y * y)
    return jnp.where(v == 0.0, jnp.float32(jnp.inf), y)


def _body(xt_hbm, anc_hbm, tg_hbm, res_hbm,
          tbuf, anc_v, cells, keptv, flagv, wmr, pcr,
          g0r, g1r, g2r, g3r, table, accv, sumbuf, blk,
          lcode, lwgt, lg0, lg1, lg2, lg3,
          s_code, s_wgt, s_g0, s_g1, s_g2, s_g3, s_part, sem):
    cid = lax.axis_index("c")
    sid = lax.axis_index("s")
    b = cid * 16 + sid

    lo = (sid & 7) * _PER
    boff = pl.multiple_of(cid * 16 + (sid & 8), 8)
    blk_cps = []
    for j in range(_PER):
        p = jnp.minimum(lo + j, jnp.int32(_NPAIR - 1))
        a_s = ((p >= _H).astype(jnp.int32) + (p >= 2 * _H) + (p >= 3 * _H)
               + (p >= 4 * _H))
        h_s = p - a_s * _H
        blk_cps.append(
            pltpu.async_copy(
                xt_hbm.at[a_s, h_s, pl.ds(0, 4), pl.ds(boff, 8)],
                blk.at[j], sem))

    tstart = b * (_NT * 5)
    tshift = tstart & 7
    tastart = pl.multiple_of(tstart - tshift, 8)
    pltpu.sync_copy(tg_hbm.at[pl.ds(tastart, _TPAD)], tbuf)
    pltpu.sync_copy(anc_hbm, anc_v.at[pl.ds(0, 2 * _A)])

    lane = lax.iota(jnp.int32, 16)
    av = anc_v[...]

    for ci in range(4):
        tvec = lane + 16 * ci
        valid = tvec < _NT
        trow = jnp.minimum(tvec, _NT - 1) * 5 + tshift
        x = plsc.load_gather(tbuf, [trow + 1])
        y = plsc.load_gather(tbuf, [trow + 2])
        w = plsc.load_gather(tbuf, [trow + 3])
        h = plsc.load_gather(tbuf, [trow + 4])
        x = jnp.where(valid, x, 0.0)
        y = jnp.where(valid, y, 0.0)
        w = jnp.where(valid, w, 0.0)
        h = jnp.where(valid, h, 0.0)
        kept = valid & ~((x == 0.0) & (y == 0.0) & (w == 0.0) & (h == 0.0))

        cxf = x * float(_W)
        cyf = y * float(_H)
        cx = cxf.astype(jnp.int32)
        cy = cyf.astype(jnp.int32)
        ctx = cxf - cx.astype(jnp.float32) - 0.5
        cty = cyf - cy.astype(jnp.float32) - 0.5
        tw = w * float(_W)
        th = h * float(_H)
        t_area = tw * th

        best = jnp.full((16,), -1.0, jnp.float32)
        bidx = jnp.zeros((16,), jnp.int32)
        tx0 = ctx - tw * 0.5
        tx1 = ctx + tw * 0.5
        ty0 = cty - th * 0.5
        ty1 = cty + th * 0.5
        for a in range(_A):
            aw = av[2 * a]
            ah = av[2 * a + 1]
            aw2 = aw * 0.5
            ah2 = ah * 0.5
            x0 = jnp.maximum(tx0, -aw2)
            x1 = jnp.minimum(tx1, aw2)
            y0 = jnp.maximum(ty0, -ah2)
            y1 = jnp.minimum(ty1, ah2)
            ivl = (x0 < x1) & (y0 < y1)
            inter = jnp.where(ivl, (x1 - x0) * (y1 - y0), 0.0)
            iou = inter / (t_area + aw * ah - inter)
            upd = iou > best
            best = jnp.where(upd, iou, best)
            bidx = jnp.where(upd, jnp.int32(a), bidx)

        flagged = kept & (best > _THRESH)
        cell = cy * _W + cx
        sl = pl.ds(16 * ci, 16)
        cells[sl] = cell
        keptv[sl] = kept.astype(jnp.int32)
        flagv[sl] = flagged.astype(jnp.int32)
        i_row = cell * _A + bidx
        a2 = ((i_row.astype(jnp.float32) + 0.5)
              * (1.0 / float(_H * _W))).astype(jnp.int32)
        rem = i_row - a2 * (_H * _W)
        h2 = ((rem.astype(jnp.float32) + 0.5)
              * (1.0 / float(_W))).astype(jnp.int32)
        w2 = rem - h2 * _W
        pcr[sl] = (a2 * _H + h2) * 32 + w2
        g0r[sl] = cxf
        g1r[sl] = cyf
        g2r[sl] = tw
        g3r[sl] = th

    for ci in range(4):
        sl = pl.ds(16 * ci, 16)
        cvec = cells[sl]
        kvec = keptv[sl]
        addrs = jnp.where(kvec > 0, cvec, jnp.int32(_H * _W))
        tvec = lane + 16 * ci
        nt_here = min(16, _NT - 16 * ci)
        for j in range(nt_here):
            plsc.store_scatter(table, [addrs], tvec, mask=lane == j)

    cnt = jnp.zeros((16,), jnp.float32)
    for ci in range(4):
        sl = pl.ds(16 * ci, 16)
        tvec = lane + 16 * ci
        winner = plsc.load_gather(table, [cells[sl]])
        wm = (flagv[sl] > 0) & (winner == tvec)
        wmr[sl] = wm.astype(jnp.int32)
        cnt = cnt + jnp.where(wm, 1.0, 0.0)
    n2 = jnp.sum(cnt)
    recip = 1.0 / (2.0 * (jnp.zeros((16,), jnp.float32) + n2))
    for ci in range(4):
        sl = pl.ds(16 * ci, 16)
        wm = wmr[sl] > 0
        pcr[sl] = jnp.where(wm, pcr[sl], 0)
        flagv[sl] = plsc.bitcast(jnp.where(wm, recip, 0.0), jnp.int32)

    psl = pl.ds(sid * _NTP, _NTP)
    pltpu.sync_copy(pcr, s_code.at[psl])
    pltpu.sync_copy(flagv, s_wgt.at[psl])
    pltpu.sync_copy(g0r, s_g0.at[psl])
    pltpu.sync_copy(g1r, s_g1.at[psl])
    pltpu.sync_copy(g2r, s_g2.at[psl])
    pltpu.sync_copy(g3r, s_g3.at[psl])
    plsc.subcore_barrier()

    for cp in blk_cps:
        cp.wait()

    goff = pl.multiple_of((sid & 8) * _NTP, 8)
    pltpu.sync_copy(s_code.at[pl.ds(goff, 8 * _NTP)], lcode)
    pltpu.sync_copy(s_wgt.at[pl.ds(goff, 8 * _NTP)], lwgt)
    pltpu.sync_copy(s_g0.at[pl.ds(goff, 8 * _NTP)], lg0)
    pltpu.sync_copy(s_g1.at[pl.ds(goff, 8 * _NTP)], lg1)
    pltpu.sync_copy(s_g2.at[pl.ds(goff, 8 * _NTP)], lg2)
    pltpu.sync_copy(s_g3.at[pl.ds(goff, 8 * _NTP)], lg3)

    contrib = jnp.zeros((16,), jnp.float32)
    for r in range(32):
        sl = pl.ds(16 * r, 16)
        code = lcode[sl]
        wgt = plsc.bitcast(lwgt[sl], jnp.float32)
        pairv = code >> 5
        cxv = code & 31
        rel = pairv - lo
        mine = (wgt > 0.0) & (rel >= 0) & (rel < _PER)
        relc = jnp.clip(rel, 0, _PER - 1)
        brel = (lane + 16 * r) >> 6
        p0 = plsc.load_gather(blk, [relc, jnp.full((16,), 0, jnp.int32),
                                    brel, cxv])
        p1 = plsc.load_gather(blk, [relc, jnp.full((16,), 1, jnp.int32),
                                    brel, cxv])
        p2 = plsc.load_gather(blk, [relc, jnp.full((16,), 2, jnp.int32),
                                    brel, cxv])
        p3 = plsc.load_gather(blk, [relc, jnp.full((16,), 3, jnp.int32),
                                    brel, cxv])
        d0 = p0 - lg0[sl]
        d1 = p1 - lg1[sl]
        d2 = _rsqrt(p2) - _rsqrt(lg2[sl])
        d3 = _rsqrt(p3) - _rsqrt(lg3[sl])
        ssq = d0 * d0 + d1 * d1 + d2 * d2 + d3 * d3
        contrib = contrib + jnp.where(mine, wgt * ssq, 0.0)

    part = jnp.sum(contrib)
    accv[...] = jnp.where(lane == 0, part, 0.0)
    pltpu.sync_copy(accv, s_part.at[pl.ds(sid * 16, 16)])
    plsc.subcore_barrier()

    @pl.when(sid == 0)
    def _():
        pltpu.sync_copy(s_part, sumbuf)
        acc = sumbuf[pl.ds(0, 16)]
        for i in range(1, 16):
            acc = acc + sumbuf[pl.ds(16 * i, 16)]
        accv[...] = acc
        pltpu.sync_copy(accv, res_hbm.at[cid])


def kernel(output, anchors, targets):
    xt = output.transpose(1, 2, 4, 0, 3)
    tg1d = targets.reshape(-1)
    anc1d = anchors.reshape(-1)
    mesh = plsc.VectorSubcoreMesh(core_axis_name="c", subcore_axis_name="s")
    k = pl.kernel(
        _body,
        mesh=mesh,
        compiler_params=pltpu.CompilerParams(needs_layout_passes=False),
        out_type=jax.ShapeDtypeStruct((2, 16), jnp.float32),
        scratch_types=[
            pltpu.VMEM((_TPAD,), jnp.float32),
            pltpu.VMEM((16,), jnp.float32),
            pltpu.VMEM((_NTP,), jnp.int32),
            pltpu.VMEM((_NTP,), jnp.int32),
            pltpu.VMEM((_NTP,), jnp.int32),
            pltpu.VMEM((_NTP,), jnp.int32),
            pltpu.VMEM((_NTP,), jnp.int32),
            pltpu.VMEM((_NTP,), jnp.float32),
            pltpu.VMEM((_NTP,), jnp.float32),
            pltpu.VMEM((_NTP,), jnp.float32),
            pltpu.VMEM((_NTP,), jnp.float32),
            pltpu.VMEM((_H * _W + 8,), jnp.int32),
            pltpu.VMEM((16,), jnp.float32),
            pltpu.VMEM((256,), jnp.float32),
            pltpu.VMEM((_PER, 4, 8, _W), jnp.float32),
            pltpu.VMEM((8 * _NTP,), jnp.int32),
            pltpu.VMEM((8 * _NTP,), jnp.int32),
            pltpu.VMEM((8 * _NTP,), jnp.float32),
            pltpu.VMEM((8 * _NTP,), jnp.float32),
            pltpu.VMEM((8 * _NTP,), jnp.float32),
            pltpu.VMEM((8 * _NTP,), jnp.float32),
            pltpu.VMEM_SHARED((16 * _NTP,), jnp.int32),
            pltpu.VMEM_SHARED((16 * _NTP,), jnp.int32),
            pltpu.VMEM_SHARED((16 * _NTP,), jnp.float32),
            pltpu.VMEM_SHARED((16 * _NTP,), jnp.float32),
            pltpu.VMEM_SHARED((16 * _NTP,), jnp.float32),
            pltpu.VMEM_SHARED((16 * _NTP,), jnp.float32),
            pltpu.VMEM_SHARED((256,), jnp.float32),
            pltpu.SemaphoreType.DMA,
        ],
    )
    res = k(xt, anc1d, tg1d)
    return (res[0, 0] + res[1, 0]) / jnp.float32(_B)
